# Initial kernel scaffold; baseline (speedup 1.0000x reference)
import jax
import jax.numpy as jnp
from jax.experimental import pallas as pl


def kernel(x, edge_index, W1, b1, Wmu, bmu, Wls, bls):
    raise NotImplementedError("write your pallas kernel here")


# speedup vs baseline: 10.0128x; 10.0128x over previous
import functools

import jax
import jax.numpy as jnp
from jax import lax
from jax.experimental import pallas as pl
from jax.experimental.pallas import tpu as pltpu
from jax.experimental.pallas import tpu_sc as plsc

N = 10000
NP = 10240
E = 320000
NC = 2
NS = 16
NW = NC * NS
CH = 128
EPT = NP
CPT = EPT // CH
EP = EPT * NW
RPS = NP // NS
D1 = 128
D2 = 64
DH = 256

_MESH = plsc.VectorSubcoreMesh(core_axis_name="c", subcore_axis_name="s")
_SC_PARAMS = pltpu.CompilerParams(needs_layout_passes=False)
_SC_PARAMS_LINEAR = pltpu.CompilerParams(
    needs_layout_passes=False, use_tc_tiling_on_sc=False)


def _deg_body(dst_hbm, zdeg_hbm, out_hbm, dstv, table):
    c = lax.axis_index("c")
    s = lax.axis_index("s")
    w = s * NC + c
    pltpu.sync_copy(zdeg_hbm, table)
    pltpu.sync_copy(dst_hbm.at[pl.ds(w * EPT, EPT)], dstv)
    ones = jnp.ones((16,), jnp.float32)

    def step(i, carry):
        idx = dstv[pl.ds(i * 16, 16)]
        plsc.addupdate_scatter(table, [idx], ones)
        return carry

    lax.fori_loop(0, EPT // 16, step, 0)
    pltpu.sync_copy(table, out_hbm.at[w])


@functools.partial(jax.jit, static_argnames=())
def _deg(dst_pad, zdeg):
    k = pl.kernel(
        _deg_body,
        out_type=jax.ShapeDtypeStruct((NW, NP), jnp.float32),
        mesh=_MESH,
        scratch_types=[
            pltpu.VMEM((EPT,), jnp.int32),
            pltpu.VMEM((NP,), jnp.float32),
        ],
        compiler_params=_SC_PARAMS,
    )
    return k(dst_pad, zdeg)


def _prop_body(D, src_hbm, dst_hbm, ytab_hbm, zrow_hbm, out_hbm,
               srcv, dstv, rows, acc, sem):
    c = lax.axis_index("c")
    s = lax.axis_index("s")
    w = s * NC + c
    pltpu.sync_copy(zrow_hbm, rows)
    for j in range(RPS // CH):
        pltpu.sync_copy(rows, acc.at[pl.ds(s * RPS + j * CH, CH)])
    plsc.subcore_barrier()

    def step(i, carry):
        base = (w * CPT + i) * CH
        pltpu.sync_copy(src_hbm.at[pl.ds(base, CH)], srcv)
        pltpu.sync_copy(dst_hbm.at[pl.ds(base, CH)], dstv)
        pltpu.async_copy(ytab_hbm.at[srcv], rows, sem).wait()
        pltpu.sync_copy(rows, acc.at[dstv], add=True)
        return carry

    lax.fori_loop(0, CPT, step, 0)
    plsc.subcore_barrier()
    for j in range(RPS // CH):
        sl = pl.ds(s * RPS + j * CH, CH)
        pltpu.sync_copy(acc.at[sl], rows)
        pltpu.sync_copy(rows, out_hbm.at[c, sl])


def _prop(src_pad, dst_pad, ytab, zrow, D):
    k = pl.kernel(
        functools.partial(_prop_body, D),
        out_type=jax.ShapeDtypeStruct((NC, NP, D), jnp.float32),
        mesh=_MESH,
        scratch_types=[
            pltpu.VMEM((CH,), jnp.int32),
            pltpu.VMEM((CH,), jnp.int32),
            pltpu.VMEM((CH, D), jnp.float32),
            pltpu.VMEM_SHARED((NP, D), jnp.float32),
            pltpu.SemaphoreType.DMA,
        ],
        compiler_params=_SC_PARAMS_LINEAR,
    )
    return k(src_pad, dst_pad, ytab, zrow)


def _prep_body(x_ref, degp_ref, y1_ref, dinv_ref):
    deg = jnp.sum(degp_ref[...], axis=0) + 1.0
    di = lax.rsqrt(deg)
    dinv_ref[...] = di
    y1_ref[...] = x_ref[...] * di


def _prep(x_pad, degp_col):
    return pl.pallas_call(
        _prep_body,
        grid=(NP // 128,),
        in_specs=[
            pl.BlockSpec((128, D1), lambda g: (g, 0)),
            pl.BlockSpec((NW, 128, 1), lambda g: (0, g, 0)),
        ],
        out_specs=[
            pl.BlockSpec((128, D1), lambda g: (g, 0)),
            pl.BlockSpec((128, 1), lambda g: (g, 0)),
        ],
        out_shape=[
            jax.ShapeDtypeStruct((NP, D1), jnp.float32),
            jax.ShapeDtypeStruct((NP, 1), jnp.float32),
        ],
    )(x_pad, degp_col)


def _mlp_body(s1_ref, y1_ref, dinv_ref, w1_ref, b1_ref, wc_ref, y2_ref):
    di = dinv_ref[...]
    agg = di * (s1_ref[0] + s1_ref[1] + y1_ref[...])
    h = jnp.dot(agg, w1_ref[...], preferred_element_type=jnp.float32)
    h = jnp.maximum(h + b1_ref[...], 0.0)
    g = jnp.dot(h, wc_ref[...], preferred_element_type=jnp.float32)
    y2_ref[...] = di * g


def _mlp(s1p, y1, dinv, W1, b1r, Wc):
    return pl.pallas_call(
        _mlp_body,
        grid=(NP // 128,),
        in_specs=[
            pl.BlockSpec((NC, 128, D1), lambda g: (0, g, 0)),
            pl.BlockSpec((128, D1), lambda g: (g, 0)),
            pl.BlockSpec((128, 1), lambda g: (g, 0)),
            pl.BlockSpec((D1, DH), lambda g: (0, 0)),
            pl.BlockSpec((1, DH), lambda g: (0, 0)),
            pl.BlockSpec((DH, D2), lambda g: (0, 0)),
        ],
        out_specs=pl.BlockSpec((128, D2), lambda g: (g, 0)),
        out_shape=jax.ShapeDtypeStruct((NP, D2), jnp.float32),
    )(s1p, y1, dinv, W1, b1r, Wc)


def _final_body(s2_ref, y2_ref, dinv_ref, bc_ref, o_ref):
    di = dinv_ref[...]
    o_ref[...] = di * (s2_ref[0] + s2_ref[1] + y2_ref[...]) + bc_ref[...]


def _final(s2p, y2, dinv, bcr):
    return pl.pallas_call(
        _final_body,
        grid=(NP // 128,),
        in_specs=[
            pl.BlockSpec((NC, 128, D2), lambda g: (0, g, 0)),
            pl.BlockSpec((128, D2), lambda g: (g, 0)),
            pl.BlockSpec((128, 1), lambda g: (g, 0)),
            pl.BlockSpec((1, D2), lambda g: (0, 0)),
        ],
        out_specs=pl.BlockSpec((128, D2), lambda g: (g, 0)),
        out_shape=jax.ShapeDtypeStruct((NP, D2), jnp.float32),
    )(s2p, y2, dinv, bcr)


def kernel(x, edge_index, W1, b1, Wmu, bmu, Wls, bls):
    src = edge_index[0]
    dst = edge_index[1]
    pad = jnp.full((EP - E,), N, dtype=jnp.int32)
    src_pad = jnp.concatenate([src, pad])
    dst_pad = jnp.concatenate([dst, pad])
    x_pad = jnp.concatenate(
        [x, jnp.zeros((NP - N, D1), jnp.float32)], axis=0)

    zdeg = jnp.zeros((NP,), jnp.float32)
    zrow1 = jnp.zeros((CH, D1), jnp.float32)
    zrow2 = jnp.zeros((CH, D2), jnp.float32)

    degp = _deg(dst_pad, zdeg)
    degp_col = degp.reshape(NW, NP, 1)

    y1, dinv = _prep(x_pad, degp_col)
    s1p = _prop(src_pad, dst_pad, y1, zrow1, D1)

    Wc = jnp.concatenate([Wmu, Wls], axis=1)
    bc = jnp.concatenate([bmu, bls]).reshape(1, D2)
    y2 = _mlp(s1p, y1, dinv, W1, b1.reshape(1, DH), Wc)

    s2p = _prop(src_pad, dst_pad, y2, zrow2, D2)
    out = _final(s2p, y2, dinv, bc)

    mu = out[:N, :32]
    logstd = out[:N, 32:]
    return (mu, logstd)


# speedup vs baseline: 12.4435x; 1.2428x over previous
import functools

import jax
import jax.numpy as jnp
from jax import lax
from jax.experimental import pallas as pl
from jax.experimental.pallas import tpu as pltpu
from jax.experimental.pallas import tpu_sc as plsc

N = 10000
NP = 10240
E = 320000
NC = 2
NS = 16
NW = NC * NS
CH = 128
EPT = NP
CPT = EPT // CH
EP = EPT * NW
RPS = NP // NS
D1 = 128
D2 = 64
DH = 256

_MESH = plsc.VectorSubcoreMesh(core_axis_name="c", subcore_axis_name="s")
_SC_PARAMS = pltpu.CompilerParams(needs_layout_passes=False)
_SC_PARAMS_LINEAR = pltpu.CompilerParams(
    needs_layout_passes=False, use_tc_tiling_on_sc=False)


def _deg_body(dst_hbm, zdeg_hbm, out_hbm, dstv, table):
    c = lax.axis_index("c")
    s = lax.axis_index("s")
    w = s * NC + c
    pltpu.sync_copy(zdeg_hbm, table)
    pltpu.sync_copy(dst_hbm.at[pl.ds(w * EPT, EPT)], dstv)
    ones = jnp.ones((16,), jnp.float32)

    def step(i, carry):
        idx = dstv[pl.ds(i * 16, 16)]
        plsc.addupdate_scatter(table, [idx], ones)
        return carry

    lax.fori_loop(0, EPT // 16, step, 0)
    pltpu.sync_copy(table, out_hbm.at[w])


@functools.partial(jax.jit, static_argnames=())
def _deg(dst_pad, zdeg):
    k = pl.kernel(
        _deg_body,
        out_type=jax.ShapeDtypeStruct((NW, NP), jnp.float32),
        mesh=_MESH,
        scratch_types=[
            pltpu.VMEM((EPT,), jnp.int32),
            pltpu.VMEM((NP,), jnp.float32),
        ],
        compiler_params=_SC_PARAMS,
    )
    return k(dst_pad, zdeg)


def _prop_body(D, src_hbm, dst_hbm, ytab_hbm, zero_hbm, out_hbm,
               src0, dst0, src1, dst1, rows0, rows1, acc, sem0, sem1):
    c = lax.axis_index("c")
    s = lax.axis_index("s")
    w = s * NC + c
    pltpu.sync_copy(zero_hbm, rows0)
    for j in range(RPS // CH):
        pltpu.sync_copy(rows0, acc.at[pl.ds(s * RPS + j * CH, CH)])
    plsc.subcore_barrier()

    def gather(i, srcv, rows, sem):
        base = (w * CPT + i) * CH
        pltpu.sync_copy(src_hbm.at[pl.ds(base, CH)], srcv)
        pltpu.async_copy(ytab_hbm.at[srcv], rows, sem)

    def scatter(i, srcv, dstv, rows, sem):
        base = (w * CPT + i) * CH
        pltpu.sync_copy(dst_hbm.at[pl.ds(base, CH)], dstv)
        pltpu.make_async_copy(ytab_hbm.at[srcv], rows, sem).wait()
        pltpu.sync_copy(rows, acc.at[dstv], add=True)

    gather(0, src0, rows0, sem0)

    def step(j, carry):
        gather(2 * j + 1, src1, rows1, sem1)
        scatter(2 * j, src0, dst0, rows0, sem0)
        gather(2 * j + 2, src0, rows0, sem0)
        scatter(2 * j + 1, src1, dst1, rows1, sem1)
        return carry

    lax.fori_loop(0, CPT // 2 - 1, step, 0)
    gather(CPT - 1, src1, rows1, sem1)
    scatter(CPT - 2, src0, dst0, rows0, sem0)
    scatter(CPT - 1, src1, dst1, rows1, sem1)
    plsc.subcore_barrier()
    for j in range(RPS // CH):
        sl = pl.ds(s * RPS + j * CH, CH)
        pltpu.sync_copy(acc.at[sl], rows0)
        pltpu.sync_copy(rows0, out_hbm.at[c, sl])


def _prop(src_pad, dst_pad, ytab, zero, D):
    k = pl.kernel(
        functools.partial(_prop_body, D),
        out_type=jax.ShapeDtypeStruct((NC, NP, D), jnp.float32),
        mesh=_MESH,
        scratch_types=[
            pltpu.VMEM((CH,), jnp.int32),
            pltpu.VMEM((CH,), jnp.int32),
            pltpu.VMEM((CH,), jnp.int32),
            pltpu.VMEM((CH,), jnp.int32),
            pltpu.VMEM((CH, D), jnp.float32),
            pltpu.VMEM((CH, D), jnp.float32),
            pltpu.VMEM_SHARED((NP, D), jnp.float32),
            pltpu.SemaphoreType.DMA,
            pltpu.SemaphoreType.DMA,
        ],
        compiler_params=_SC_PARAMS_LINEAR,
    )
    return k(src_pad, dst_pad, ytab, zero)


def _prep_body(x_ref, degp_ref, y1_ref, dinv_ref):
    deg = jnp.sum(degp_ref[...], axis=0) + 1.0
    di = lax.rsqrt(deg)
    dinv_ref[...] = di
    y1_ref[...] = x_ref[...] * di


def _prep(x_pad, degp_col):
    return pl.pallas_call(
        _prep_body,
        grid=(NP // 128,),
        in_specs=[
            pl.BlockSpec((128, D1), lambda g: (g, 0)),
            pl.BlockSpec((NW, 128, 1), lambda g: (0, g, 0)),
        ],
        out_specs=[
            pl.BlockSpec((128, D1), lambda g: (g, 0)),
            pl.BlockSpec((128, 1), lambda g: (g, 0)),
        ],
        out_shape=[
            jax.ShapeDtypeStruct((NP, D1), jnp.float32),
            jax.ShapeDtypeStruct((NP, 1), jnp.float32),
        ],
    )(x_pad, degp_col)


def _mlp_body(s1_ref, y1_ref, dinv_ref, w1_ref, b1_ref, wc_ref, y2_ref):
    di = dinv_ref[...]
    agg = di * (s1_ref[0] + s1_ref[1] + y1_ref[...])
    h = jnp.dot(agg, w1_ref[...], preferred_element_type=jnp.float32)
    h = jnp.maximum(h + b1_ref[...], 0.0)
    g = jnp.dot(h, wc_ref[...], preferred_element_type=jnp.float32)
    y2_ref[...] = di * g


def _mlp(s1p, y1, dinv, W1, b1r, Wc):
    return pl.pallas_call(
        _mlp_body,
        grid=(NP // 128,),
        in_specs=[
            pl.BlockSpec((NC, 128, D1), lambda g: (0, g, 0)),
            pl.BlockSpec((128, D1), lambda g: (g, 0)),
            pl.BlockSpec((128, 1), lambda g: (g, 0)),
            pl.BlockSpec((D1, DH), lambda g: (0, 0)),
            pl.BlockSpec((1, DH), lambda g: (0, 0)),
            pl.BlockSpec((DH, D2), lambda g: (0, 0)),
        ],
        out_specs=pl.BlockSpec((128, D2), lambda g: (g, 0)),
        out_shape=jax.ShapeDtypeStruct((NP, D2), jnp.float32),
    )(s1p, y1, dinv, W1, b1r, Wc)


def _final_body(s2_ref, y2_ref, dinv_ref, bc_ref, o_ref):
    di = dinv_ref[...]
    o_ref[...] = di * (s2_ref[0] + s2_ref[1] + y2_ref[...]) + bc_ref[...]


def _final(s2p, y2, dinv, bcr):
    return pl.pallas_call(
        _final_body,
        grid=(NP // 128,),
        in_specs=[
            pl.BlockSpec((NC, 128, D2), lambda g: (0, g, 0)),
            pl.BlockSpec((128, D2), lambda g: (g, 0)),
            pl.BlockSpec((128, 1), lambda g: (g, 0)),
            pl.BlockSpec((1, D2), lambda g: (0, 0)),
        ],
        out_specs=pl.BlockSpec((128, D2), lambda g: (g, 0)),
        out_shape=jax.ShapeDtypeStruct((NP, D2), jnp.float32),
    )(s2p, y2, dinv, bcr)


def kernel(x, edge_index, W1, b1, Wmu, bmu, Wls, bls):
    src = edge_index[0]
    dst = edge_index[1]
    pad = jnp.full((EP - E,), N, dtype=jnp.int32)
    src_pad = jnp.concatenate([src, pad])
    dst_pad = jnp.concatenate([dst, pad])
    src3 = src_pad.reshape(NW, CPT, CH)
    dst3 = dst_pad.reshape(NW, CPT, CH)
    x_pad = jnp.concatenate(
        [x, jnp.zeros((NP - N, D1), jnp.float32)], axis=0)

    zdeg = jnp.zeros((NP,), jnp.float32)
    zrow1 = jnp.zeros((CH, D1), jnp.float32)
    zrow2 = jnp.zeros((CH, D2), jnp.float32)

    degp = _deg(dst_pad, zdeg)
    degp_col = degp.reshape(NW, NP, 1)

    y1, dinv = _prep(x_pad, degp_col)
    s1p = _prop(src_pad, dst_pad, y1, zrow1, D1)

    Wc = jnp.concatenate([Wmu, Wls], axis=1)
    bc = jnp.concatenate([bmu, bls]).reshape(1, D2)
    y2 = _mlp(s1p, y1, dinv, W1, b1.reshape(1, DH), Wc)

    s2p = _prop(src_pad, dst_pad, y2, zrow2, D2)
    out = _final(s2p, y2, dinv, bc)

    mu = out[:N, :32]
    logstd = out[:N, 32:]
    return (mu, logstd)
